# Initial kernel scaffold; baseline (speedup 1.0000x reference)
import jax
import jax.numpy as jnp
from jax.experimental import pallas as pl


def kernel(x, edge_index, i, params):
    raise NotImplementedError("write your pallas kernel here")


# speedup vs baseline: 1.5825x; 1.5825x over previous
import functools

import jax
import jax.numpy as jnp
import numpy as np
from jax import lax
from jax.experimental import pallas as pl
from jax.experimental.pallas import tpu as pltpu
from jax.experimental.pallas import tpu_sc as plsc

N = 10000
E = 320000
F = 128
H = 64
G = 64

NC = 2
NS = 16
NW = NC * NS
CHUNK = 128
CH = 79
EW = CH * CHUNK
EP = NW * EW
NP = 10240
NB = NP // 128
EB = 512
NEB = EP // EB
RPS = NP // NS
H0OFF = 48

_INV = float(1.0 / np.sqrt(1.0 + 1e-3))


def _full_spec(shape):
    nd = len(shape)
    return pl.BlockSpec(shape, lambda b: (0,) * nd)


def _node_enc_body(xb, wx1, bx1, wx2, bx2, qmat, sel4, sel32, selh0, t_out):
    x = xb[...]
    h = x @ wx1[...] + bx1[...]
    h0 = jnp.maximum(h @ wx2[...] + bx2[...], 0.0)
    q = x @ qmat[...]
    t_out[...] = x @ sel4[...] + q @ sel32[...] + h0 @ selh0[...]


def _edge_enc_body(trb, tsb, selp, wvp, wd, be1, we2, be2, sel72, mrow, ee_out):
    dx = trb[...] - tsb[...]
    lane = lax.broadcasted_iota(jnp.int32, dx.shape, 1)
    dxm = jnp.where(lane < 3, dx, 0.0)
    ss = jnp.sum(dxm * dxm, axis=1, keepdims=True)
    dist = jnp.sqrt(ss)
    safe = jnp.where(dist > 0.0, dist, 1.0)
    vects = dxm / safe
    e1 = dx @ selp[...] + dist * wd[...] + vects @ wvp[...] + be1[...]
    ee = jnp.maximum(e1 @ we2[...] + be2[...], 0.0)
    col3 = jnp.sum(jnp.where(lane == 3, dx, 0.0), axis=1, keepdims=True)
    maskf = (col3 > 0.0).astype(jnp.float32)
    ee_out[...] = ee @ sel72[...] + maskf * mrow[...]


def _msg_body(grb, gsb, eeb, w1r, w1s, w1e, b1, w2, b2, m_out):
    g = grb[...] @ w1r[...] + gsb[...] @ w1s[...]
    ee = eeb[...]
    m1 = jnp.maximum(g + ee @ w1e[...] + b1[...], 0.0)
    lane = lax.broadcasted_iota(jnp.int32, ee.shape, 1)
    maskf = jnp.sum(jnp.where(lane == H, ee, 0.0), axis=1, keepdims=True)
    m_out[...] = jnp.maximum(m1 @ w2[...] + b2[...], 0.0) * maskf


def _update_body(pb, w3, b3, w4, b4, h_out):
    agg = pb[0] + pb[1]
    u = jnp.maximum(agg @ w3[...] + b3[...], 0.0)
    h_out[...] = jnp.maximum(u @ w4[...] + b4[...], 0.0)


def _pool_dec_body(hb, segb, selh, c64, seltop, e64, af, cf,
                   wd1, bd1, wd2, bd2, wd3, bd3, wd4, bd4,
                   wo1, bo1, wo2, bo2, sel7, o_out, acc):
    b = pl.program_id(0)

    @pl.when(b == 0)
    def _():
        acc[...] = jnp.zeros_like(acc)

    seg = segb[0]
    onehot = (lax.broadcasted_iota(jnp.int32, (G, 128), 0) == seg).astype(
        jnp.float32)
    hbx = hb[...] @ selh[...] + c64[...]
    acc[...] += onehot @ hbx

    @pl.when(b == NB - 1)
    def _():
        s = acc[...]
        sums = s @ seltop[...]
        cnt = s @ e64[...]
        d = (sums / jnp.maximum(cnt, 1.0)) * af[...] + cf[...]
        d = jnp.maximum(d @ wd1[...] + bd1[...], 0.0)
        d = jnp.maximum(d @ wd2[...] + bd2[...], 0.0)
        d = jnp.maximum(d @ wd3[...] + bd3[...], 0.0)
        d = jnp.maximum(d @ wd4[...] + bd4[...], 0.0)
        o = jnp.tanh(d @ wo1[...] + bo1[...])
        o = o @ wo2[...] + bo2[...]
        o_out[...] = o @ sel7[...]


def _sc_mesh():
    return plsc.VectorSubcoreMesh(core_axis_name="c", subcore_axis_name="s",
                                  num_cores=NC, num_subcores=NS)


def _make_gather2():
    @functools.partial(
        pl.kernel,
        out_type=(jax.ShapeDtypeStruct((EP, 128), jnp.float32),
                  jax.ShapeDtypeStruct((EP, 128), jnp.float32)),
        mesh=_sc_mesh(),
        scratch_types=[
            pltpu.VMEM((CHUNK,), jnp.int32),
            pltpu.VMEM((CHUNK,), jnp.int32),
            pltpu.VMEM((CHUNK, 128), jnp.float32),
            pltpu.VMEM((CHUNK, 128), jnp.float32),
            pltpu.SemaphoreType.DMA,
            pltpu.SemaphoreType.DMA,
        ],
    )
    def gather2(table, idx_a, idx_b, out_a, out_b, iva, ivb, ra, rb, sa, sb):
        wid = lax.axis_index("s") * NC + lax.axis_index("c")
        base0 = wid * EW

        def body(c, carry):
            base = base0 + c * CHUNK
            pltpu.sync_copy(idx_a.at[pl.ds(base, CHUNK)], iva)
            pltpu.sync_copy(idx_b.at[pl.ds(base, CHUNK)], ivb)
            cpa = pltpu.async_copy(table.at[iva], ra, sa)
            cpb = pltpu.async_copy(table.at[ivb], rb, sb)
            cpa.wait()
            cpb.wait()
            pltpu.sync_copy(ra, out_a.at[pl.ds(base, CHUNK)])
            pltpu.sync_copy(rb, out_b.at[pl.ds(base, CHUNK)])
            return carry

        lax.fori_loop(0, CH, body, 0)

    return gather2


def _make_scatter_add():
    @functools.partial(
        pl.kernel,
        out_type=jax.ShapeDtypeStruct((NC, NP, F), jnp.float32),
        mesh=_sc_mesh(),
        scratch_types=[
            pltpu.VMEM((CHUNK,), jnp.int32),
            pltpu.VMEM((CHUNK, F), jnp.float32),
            pltpu.VMEM_SHARED((NP, F), jnp.float32),
        ],
    )
    def scatter_add(m2, ridx, zeros_hbm, out, idxv, rows, acc):
        cid = lax.axis_index("c")
        sid = lax.axis_index("s")
        pltpu.sync_copy(zeros_hbm.at[pl.ds(sid * RPS, RPS)],
                        acc.at[pl.ds(sid * RPS, RPS)])
        plsc.subcore_barrier()
        base0 = (cid * NS + sid) * EW

        def body(c, carry):
            base = base0 + c * CHUNK
            pltpu.sync_copy(ridx.at[pl.ds(base, CHUNK)], idxv)
            pltpu.sync_copy(m2.at[pl.ds(base, CHUNK)], rows)
            pltpu.sync_copy(rows, acc.at[idxv], add=True)
            return carry

        lax.fori_loop(0, CH, body, 0)
        plsc.subcore_barrier()
        pltpu.sync_copy(acc.at[pl.ds(sid * RPS, RPS)],
                        out.at[cid, pl.ds(sid * RPS, RPS)])

    return scatter_add


def _row(v):
    return jnp.reshape(v, (1, -1))


def kernel(x, edge_index, i, params):
    f32 = jnp.float32
    send = edge_index[0].astype(jnp.int32)
    recv = edge_index[1].astype(jnp.int32)
    seg = i.astype(jnp.int32)

    pad_e = EP - E
    send_p = jnp.concatenate([send, jnp.zeros((pad_e,), jnp.int32)])
    recv_p = jnp.concatenate([recv, jnp.zeros((pad_e,), jnp.int32)])
    xp = jnp.pad(x, ((0, NP - N), (0, 0)))
    seg_p = jnp.concatenate([seg, jnp.full((NP - N,), G, jnp.int32)])
    seg3 = seg_p.reshape(NB, 1, 128)

    wx1, bx1 = params['enc_x1']
    wx2, bx2 = params['enc_x2']
    we1, be1 = params['enc_e1']
    we2, be2 = params['enc_e2']

    qmat = jnp.zeros((F, H // 2), f32).at[3:].set(we1[0:F - 3])
    wd = _row(we1[F - 3])
    wvp = jnp.zeros((128, H // 2), f32).at[0:3].set(we1[F - 2:F + 1])
    selp = jnp.zeros((128, H // 2), f32).at[4:4 + H // 2].set(jnp.eye(H // 2))
    sel4 = jnp.zeros((F, 128), f32).at[0:4, 0:4].set(jnp.eye(4))
    sel32 = jnp.zeros((H // 2, 128), f32).at[:, 4:4 + H // 2].set(
        jnp.eye(H // 2))
    selh0 = jnp.zeros((H, 128), f32).at[:, H0OFF:H0OFF + H].set(jnp.eye(H))
    sel72 = jnp.zeros((H, H + 8), f32).at[:, 0:H].set(jnp.eye(H))
    mrow = jnp.zeros((1, H + 8), f32).at[0, H].set(1.0)

    layers = []
    for li, lp in enumerate(params['mp']):
        a = lp['bn_g'] * _INV
        c = lp['bn_b']
        w1, b1 = lp['msg1']
        off = H0OFF if li == 0 else 0
        w1r = jnp.zeros((128, 4 * H), f32).at[off:off + H].set(
            a[:, None] * w1[0:H])
        w1s = jnp.zeros((128, 4 * H), f32).at[off:off + H].set(
            a[:, None] * w1[H:2 * H])
        w1e = jnp.zeros((H + 8, 4 * H), f32).at[0:H].set(w1[2 * H:3 * H])
        b1f = _row(b1 + c @ w1[0:H] + c @ w1[H:2 * H])
        w2, b2 = lp['msg2']
        w3, b3 = lp['upd1']
        w4, b4 = lp['upd2']
        w4p = jnp.zeros((2 * H, 128), f32).at[:, 0:H].set(w4)
        b4p = jnp.zeros((1, 128), f32).at[0, 0:H].set(b4)
        layers.append((w1r, w1s, w1e, b1f, w2, _row(b2), w3, _row(b3),
                       w4p, b4p))

    af = _row(params['bnd_g'] * _INV)
    cf = _row(params['bnd_b'])
    dec = params['dec']
    wo1, bo1 = params['out1']
    wo2, bo2 = params['out2']

    selh = jnp.zeros((128, 128), f32).at[0:H, 0:H].set(jnp.eye(H))
    c64 = jnp.zeros((1, 128), f32).at[0, H].set(1.0)
    seltop = jnp.zeros((128, H), f32).at[0:H].set(jnp.eye(H))
    e64 = jnp.zeros((128, H), f32).at[H, :].set(1.0)
    sel7 = jnp.zeros((7, 128), f32).at[:, 0:7].set(jnp.eye(7))

    t_tab = pl.pallas_call(
        _node_enc_body,
        grid=(NB,),
        in_specs=[
            pl.BlockSpec((128, F), lambda b: (b, 0)),
            _full_spec((F, H // 2)), _full_spec((1, H // 2)),
            _full_spec((H // 2, H)), _full_spec((1, H)),
            _full_spec((F, H // 2)),
            _full_spec((F, 128)), _full_spec((H // 2, 128)),
            _full_spec((H, 128)),
        ],
        out_specs=pl.BlockSpec((128, 128), lambda b: (b, 0)),
        out_shape=jax.ShapeDtypeStruct((NP, 128), f32),
    )(xp, wx1, _row(bx1), wx2, _row(bx2), qmat, sel4, sel32, selh0)

    gather2 = _make_gather2()
    scatter_add = _make_scatter_add()
    zeros_acc = jnp.zeros((NP, F), f32)

    g_r, g_s = gather2(t_tab, recv_p, send_p)

    eeb = pl.pallas_call(
        _edge_enc_body,
        grid=(NEB,),
        in_specs=[
            pl.BlockSpec((EB, 128), lambda b: (b, 0)),
            pl.BlockSpec((EB, 128), lambda b: (b, 0)),
            _full_spec((128, H // 2)), _full_spec((128, H // 2)),
            _full_spec((1, H // 2)), _full_spec((1, H // 2)),
            _full_spec((H // 2, H)), _full_spec((1, H)),
            _full_spec((H, H + 8)), _full_spec((1, H + 8)),
        ],
        out_specs=pl.BlockSpec((EB, H + 8), lambda b: (b, 0)),
        out_shape=jax.ShapeDtypeStruct((EP, H + 8), f32),
    )(g_r, g_s, selp, wvp, wd, _row(be1), we2, _row(be2), sel72, mrow)

    for li, (w1r, w1s, w1e, b1f, w2, b2, w3, b3, w4p, b4p) in \
            enumerate(layers):
        if li > 0:
            g_r, g_s = gather2(h, recv_p, send_p)
        m2 = pl.pallas_call(
            _msg_body,
            grid=(NEB,),
            in_specs=[
                pl.BlockSpec((EB, 128), lambda b: (b, 0)),
                pl.BlockSpec((EB, 128), lambda b: (b, 0)),
                pl.BlockSpec((EB, H + 8), lambda b: (b, 0)),
                _full_spec((128, 4 * H)), _full_spec((128, 4 * H)),
                _full_spec((H + 8, 4 * H)), _full_spec((1, 4 * H)),
                _full_spec((4 * H, 2 * H)), _full_spec((1, 2 * H)),
            ],
            out_specs=pl.BlockSpec((EB, F), lambda b: (b, 0)),
            out_shape=jax.ShapeDtypeStruct((EP, F), f32),
        )(g_r, g_s, eeb, w1r, w1s, w1e, b1f, w2, b2)

        partial = scatter_add(m2, recv_p, zeros_acc)

        h = pl.pallas_call(
            _update_body,
            grid=(NB,),
            in_specs=[
                pl.BlockSpec((NC, 128, F), lambda b: (0, b, 0)),
                _full_spec((2 * H, 2 * H)), _full_spec((1, 2 * H)),
                _full_spec((2 * H, 128)), _full_spec((1, 128)),
            ],
            out_specs=pl.BlockSpec((128, 128), lambda b: (b, 0)),
            out_shape=jax.ShapeDtypeStruct((NP, 128), f32),
        )(partial, w3, b3, w4p, b4p)

    o_pad = pl.pallas_call(
        _pool_dec_body,
        grid=(NB,),
        in_specs=[
            pl.BlockSpec((128, 128), lambda b: (b, 0)),
            pl.BlockSpec((1, 1, 128), lambda b: (b, 0, 0)),
            _full_spec((128, 128)), _full_spec((1, 128)),
            _full_spec((128, H)), _full_spec((128, H)),
            _full_spec((1, H)), _full_spec((1, H)),
            _full_spec((H, 2 * H)), _full_spec((1, 2 * H)),
            _full_spec((2 * H, 2 * H)), _full_spec((1, 2 * H)),
            _full_spec((2 * H, 2 * H)), _full_spec((1, 2 * H)),
            _full_spec((2 * H, H)), _full_spec((1, H)),
            _full_spec((H, H // 2)), _full_spec((1, H // 2)),
            _full_spec((H // 2, 7)), _full_spec((1, 7)),
            _full_spec((7, 128)),
        ],
        out_specs=pl.BlockSpec((G, 128), lambda b: (0, 0)),
        out_shape=jax.ShapeDtypeStruct((G, 128), f32),
        scratch_shapes=[pltpu.VMEM((G, 128), f32)],
    )(h, seg3, selh, c64, seltop, e64, af, cf,
      dec[0][0], _row(dec[0][1]), dec[1][0], _row(dec[1][1]),
      dec[2][0], _row(dec[2][1]), dec[3][0], _row(dec[3][1]),
      wo1, _row(bo1), wo2, _row(bo2), sel7)

    return o_pad[:, 0:7]
